# Initial kernel scaffold; baseline (speedup 1.0000x reference)
import jax
import jax.numpy as jnp
from jax.experimental import pallas as pl


def kernel(input, target):
    raise NotImplementedError("write your pallas kernel here")


# speedup vs baseline: 2.8228x; 2.8228x over previous
import functools

import jax
import jax.numpy as jnp
from jax.experimental import pallas as pl
from jax.experimental.pallas import tpu as pltpu

_RATIO = 0.05
_R = 256


def _ohem_body(x_ref, t_ref, out_ref, loss_ref, *, nblocks, k):
    i = pl.program_id(0)
    x = x_ref[...]
    tgt = t_ref[0, 0, :]

    m = jnp.max(x, axis=1, keepdims=True)
    s = jnp.sum(jnp.exp(x - m), axis=1)
    lse = m[:, 0] + jnp.log(s)
    col = jax.lax.broadcasted_iota(jnp.int32, x.shape, 1)
    t_logit = jnp.sum(jnp.where(col == tgt[:, None], x, 0.0), axis=1)
    loss = lse - t_logit
    loss_ref[pl.ds(i, 1), :] = loss.reshape(1, -1)

    @pl.when(i == nblocks - 1)
    def _select():
        vals = loss_ref[...]
        bits = jax.lax.bitcast_convert_type(vals, jnp.int32)

        def body(j, lo):
            cand = lo + (1 << (30 - j))
            cnt = jnp.sum((bits >= cand).astype(jnp.int32))
            return jnp.where(cnt >= k, cand, lo)

        thr = jax.lax.fori_loop(0, 31, body, jnp.int32(0))
        tval = jax.lax.bitcast_convert_type(thr, jnp.float32)
        gt = bits > thr
        cnt_gt = jnp.sum(gt.astype(jnp.int32))
        sum_gt = jnp.sum(jnp.where(gt, vals, 0.0))
        out_ref[0, 0] = (sum_gt + (k - cnt_gt).astype(jnp.float32) * tval) / k


@functools.partial(jax.jit, static_argnames=("interpret",))
def kernel(input, target, interpret=False):
    n, c = input.shape
    nblocks = n // _R
    k = max(1, int(n * _RATIO))
    out = pl.pallas_call(
        functools.partial(_ohem_body, nblocks=nblocks, k=k),
        grid=(nblocks,),
        in_specs=[
            pl.BlockSpec((_R, c), lambda i: (i, 0)),
            pl.BlockSpec((1, 1, _R), lambda i: (i, 0, 0)),
        ],
        out_specs=pl.BlockSpec(memory_space=pltpu.SMEM),
        out_shape=jax.ShapeDtypeStruct((1, 1), jnp.float32),
        scratch_shapes=[pltpu.VMEM((nblocks, _R), jnp.float32)],
        compiler_params=pltpu.CompilerParams(
            dimension_semantics=("arbitrary",),
        ),
        interpret=interpret,
    )(input, target.reshape(nblocks, 1, _R))
    return out[0, 0]
